# baseline (speedup 1.0000x reference)
import functools

import jax
import jax.numpy as jnp
from jax import lax
from jax.experimental import pallas as pl
from jax.experimental.pallas import tpu as pltpu
from jax.experimental.pallas import tpu_sc as plsc

_N = 10000
_C = 128
_NCLASS = 7
_NLAYERS = 4
_E = 320000

_NP = 10240
_NPAD_ROWS = _NP - _N
_NT = 32
_K = 64
_NPH = 2
_NCHH = 80
_TOTAL = _NT * _NPH * _NCHH * _K
_KD = 128
_NCHD = 80
_RPT = _NP // 16
_BN = 512

_mesh = plsc.VectorSubcoreMesh(core_axis_name="c", subcore_axis_name="s")


@functools.partial(
    pl.kernel,
    out_type=jax.ShapeDtypeStruct((2, _NP), jnp.float32),
    mesh=_mesh,
    scratch_types=[
        pltpu.VMEM((_NCHD, _KD), jnp.int32),
        pltpu.VMEM((_KD,), jnp.float32),
        pltpu.VMEM_SHARED((_NP,), jnp.float32),
    ],
)
def _deg_kernel(dst_hbm, init_hbm, ones_hbm, deg_out, dst_v, ones_v, acc):
    c = lax.axis_index("c")
    s = lax.axis_index("s")
    wid = c * 16 + s
    r0 = s * _RPT
    pltpu.sync_copy(init_hbm.at[c, pl.ds(r0, _RPT)], acc.at[pl.ds(r0, _RPT)])
    pltpu.sync_copy(ones_hbm.at[pl.ds(0, _KD)], ones_v)
    pltpu.sync_copy(dst_hbm.at[wid], dst_v)
    plsc.subcore_barrier()

    def body(j, carry):
        pltpu.sync_copy(ones_v, acc.at[dst_v.at[j]], add=True)
        return carry

    lax.fori_loop(0, _NCHD, body, 0)
    plsc.subcore_barrier()
    pltpu.sync_copy(acc.at[pl.ds(r0, _RPT)], deg_out.at[c, pl.ds(r0, _RPT)])


@functools.partial(
    pl.kernel,
    out_type=jax.ShapeDtypeStruct((2, _NP, _C), jnp.float32),
    mesh=_mesh,
    scratch_types=[
        pltpu.VMEM((_NCHH + 2, _K), jnp.int32),
        pltpu.VMEM((_NCHH, _K), jnp.int32),
        pltpu.VMEM((_K, _C), jnp.float32),
        pltpu.VMEM((_K, _C), jnp.float32),
        pltpu.VMEM_SHARED((_NP, _C), jnp.float32),
        pltpu.SemaphoreType.DMA,
        pltpu.SemaphoreType.DMA,
        pltpu.SemaphoreType.DMA,
        pltpu.SemaphoreType.DMA,
    ],
)
def _spmm_kernel(xs_hbm, zeros_hbm, src_hbm, dst_hbm, out_hbm,
                 src_v, dst_v, rows_a, rows_b, acc,
                 gsem_a, gsem_b, ssem_a, ssem_b):
    c = lax.axis_index("c")
    s = lax.axis_index("s")
    wid = c * 16 + s
    r0 = s * _RPT

    @pl.when(c == 0)
    def _():
        pltpu.sync_copy(xs_hbm.at[pl.ds(r0, _RPT)], acc.at[pl.ds(r0, _RPT)])

    @pl.when(c == 1)
    def _():
        pltpu.sync_copy(zeros_hbm.at[pl.ds(r0, _RPT)], acc.at[pl.ds(r0, _RPT)])

    plsc.subcore_barrier()

    for h in range(_NPH):
        pltpu.sync_copy(src_hbm.at[wid, h], src_v)
        pltpu.sync_copy(dst_hbm.at[wid, h], dst_v)
        pltpu.async_copy(xs_hbm.at[src_v.at[0]], rows_a, gsem_a)
        pltpu.async_copy(xs_hbm.at[src_v.at[1]], rows_b, gsem_b)

        def body(g2, carry):
            g = g2 * 2
            pltpu.make_async_copy(
                xs_hbm.at[src_v.at[g]], rows_a, gsem_a).wait()
            pltpu.async_copy(rows_a, acc.at[dst_v.at[g]], ssem_a, add=True)
            pltpu.make_async_copy(
                xs_hbm.at[src_v.at[g + 1]], rows_b, gsem_b).wait()
            pltpu.async_copy(rows_b, acc.at[dst_v.at[g + 1]], ssem_b,
                             add=True)
            pltpu.make_async_copy(rows_a, acc.at[dst_v.at[g]],
                                  ssem_a).wait()
            pltpu.async_copy(xs_hbm.at[src_v.at[g + 2]], rows_a, gsem_a)
            pltpu.make_async_copy(rows_b, acc.at[dst_v.at[g + 1]],
                                  ssem_b).wait()
            pltpu.async_copy(xs_hbm.at[src_v.at[g + 3]], rows_b, gsem_b)
            return carry

        lax.fori_loop(0, _NCHH // 2, body, 0)
        pltpu.make_async_copy(xs_hbm.at[src_v.at[0]], rows_a, gsem_a).wait()
        pltpu.make_async_copy(xs_hbm.at[src_v.at[1]], rows_b, gsem_b).wait()

    plsc.subcore_barrier()
    pltpu.sync_copy(acc.at[pl.ds(r0, _RPT)], out_hbm.at[c, pl.ds(r0, _RPT)])


def _pre_body(deg_ref, x_ref, dinv_ref, xs_ref):
    d = deg_ref[:, 0] + deg_ref[:, 1]
    dinv = jnp.where(d > 0, lax.rsqrt(jnp.maximum(d, 1e-12)), 0.0)
    dinv_ref[:, 0] = dinv
    xs_ref[...] = x_ref[...] * dinv[:, None]


def _layer_body(wb_ref, s_ref, dinv_ref, x_ref, xs_ref):
    w = wb_ref[0]
    b = wb_ref[1]
    dinv = dinv_ref[:, 0]
    x = jnp.maximum((s_ref[0] + s_ref[1]) * dinv[:, None] * w + b, 0.0)
    x_ref[...] = x
    xs_ref[...] = x * dinv[:, None]


def _readout_body(ro_ref, x_ref, out3_ref):
    x = x_ref[...]
    out3_ref[...] = (x[:, :, None] * ro_ref[0][None, None, :]
                     + ro_ref[1][None, None, :])


def kernel(x, edge_index, conv_W, conv_b, ro_W, ro_b):
    f32 = jnp.float32
    pad = (jnp.arange(_TOTAL - _E, dtype=jnp.int32) % _NPAD_ROWS) + _N
    src_flat = jnp.concatenate([edge_index[0], pad])
    dst_flat = jnp.concatenate([edge_index[1], pad])
    src_p = src_flat.reshape(_NT, _NPH, _NCHH, _K)
    dst_p = dst_flat.reshape(_NT, _NPH, _NCHH, _K)
    ring_pad = ((jnp.arange(_NT * _NPH * 2 * _K, dtype=jnp.int32)
                 % _NPAD_ROWS) + _N).reshape(_NT, _NPH, 2, _K)
    src_p = jnp.concatenate([src_p, ring_pad], axis=2)
    dst_deg = dst_flat.reshape(_NT, _NCHD, _KD)
    x_pad = jnp.zeros((_NP, _C), f32).at[:_N].set(x)
    zeros_big = jnp.zeros((_NP, _C), f32)
    ones_np = jnp.ones((_NP,), f32)
    deg_init = jnp.stack(
        [jnp.concatenate([jnp.ones((_N,), f32), jnp.zeros((_NPAD_ROWS,), f32)]),
         jnp.zeros((_NP,), f32)])
    wb = jnp.stack([conv_W[0, 0], conv_b[0]])
    ro = jnp.concatenate([ro_W, ro_b[None, :]], axis=0)

    deg2 = _deg_kernel(dst_deg, deg_init, ones_np)
    deg_t = deg2.T

    grid = (_NP // _BN,)
    dinv, xs = pl.pallas_call(
        _pre_body,
        grid=grid,
        in_specs=[
            pl.BlockSpec((_BN, 2), lambda i: (i, 0)),
            pl.BlockSpec((_BN, _C), lambda i: (i, 0)),
        ],
        out_specs=[
            pl.BlockSpec((_BN, 1), lambda i: (i, 0)),
            pl.BlockSpec((_BN, _C), lambda i: (i, 0)),
        ],
        out_shape=[
            jax.ShapeDtypeStruct((_NP, 1), f32),
            jax.ShapeDtypeStruct((_NP, _C), f32),
        ],
    )(deg_t, x_pad)

    layer_call = pl.pallas_call(
        _layer_body,
        grid=grid,
        in_specs=[
            pl.BlockSpec(memory_space=pltpu.SMEM),
            pl.BlockSpec((2, _BN, _C), lambda i: (0, i, 0)),
            pl.BlockSpec((_BN, 1), lambda i: (i, 0)),
        ],
        out_specs=[
            pl.BlockSpec((_BN, _C), lambda i: (i, 0)),
            pl.BlockSpec((_BN, _C), lambda i: (i, 0)),
        ],
        out_shape=[
            jax.ShapeDtypeStruct((_NP, _C), f32),
            jax.ShapeDtypeStruct((_NP, _C), f32),
        ],
    )

    xs_list = []
    for _ in range(_NLAYERS):
        s_part = _spmm_kernel(xs, zeros_big, src_p, dst_p)
        xlayer, xs = layer_call(wb, s_part, dinv)
        xs_list.append(xlayer)

    _BR = 80
    out = pl.pallas_call(
        _readout_body,
        grid=(_N // _BR,),
        in_specs=[
            pl.BlockSpec((2, _NCLASS), lambda i: (0, 0)),
            pl.BlockSpec((_BR, _C), lambda i: (i, 0)),
        ],
        out_specs=pl.BlockSpec((_BR, _C, _NCLASS), lambda i: (i, 0, 0)),
        out_shape=jax.ShapeDtypeStruct((_N, _C, _NCLASS), f32),
    )(ro, xs_list[-1])

    x_all = jnp.stack([x] + [xl[:_N] for xl in xs_list], axis=1)
    return (out, x_all)


# speedup vs baseline: 1.5648x; 1.5648x over previous
import functools

import jax
import jax.numpy as jnp
from jax import lax
from jax.experimental import pallas as pl
from jax.experimental.pallas import tpu as pltpu
from jax.experimental.pallas import tpu_sc as plsc

_N = 10000
_C = 128
_NCLASS = 7
_NLAYERS = 4
_E = 320000

_NP = 10240
_NPAD_ROWS = _NP - _N
_NT = 32
_K = 64
_NPH = 2
_NCHH = 80
_TOTAL = _NT * _NPH * _NCHH * _K
_KD = 128
_NCHD = 80
_RPT = _NP // 16
_BN = 512

_mesh = plsc.VectorSubcoreMesh(core_axis_name="c", subcore_axis_name="s")


@functools.partial(
    pl.kernel,
    out_type=jax.ShapeDtypeStruct((2, _NP), jnp.float32),
    mesh=_mesh,
    scratch_types=[
        pltpu.VMEM((_NCHD, _KD), jnp.int32),
        pltpu.VMEM((_KD,), jnp.float32),
        pltpu.VMEM_SHARED((_NP,), jnp.float32),
    ],
)
def _deg_kernel(dst_hbm, init_hbm, ones_hbm, deg_out, dst_v, ones_v, acc):
    c = lax.axis_index("c")
    s = lax.axis_index("s")
    wid = c * 16 + s
    r0 = s * _RPT
    pltpu.sync_copy(init_hbm.at[c, pl.ds(r0, _RPT)], acc.at[pl.ds(r0, _RPT)])
    pltpu.sync_copy(ones_hbm.at[pl.ds(0, _KD)], ones_v)
    pltpu.sync_copy(dst_hbm.at[wid], dst_v)
    plsc.subcore_barrier()

    def body(j, carry):
        pltpu.sync_copy(ones_v, acc.at[dst_v.at[j]], add=True)
        return carry

    lax.fori_loop(0, _NCHD, body, 0)
    plsc.subcore_barrier()
    pltpu.sync_copy(acc.at[pl.ds(r0, _RPT)], deg_out.at[c, pl.ds(r0, _RPT)])


@functools.partial(
    pl.kernel,
    out_type=jax.ShapeDtypeStruct((2, _NP, _C), jnp.float32),
    mesh=_mesh,
    scratch_types=[
        pltpu.VMEM((_NCHH + 2, _K), jnp.int32),
        pltpu.VMEM((_NCHH, _K), jnp.int32),
        pltpu.VMEM((_K, _C), jnp.float32),
        pltpu.VMEM((_K, _C), jnp.float32),
        pltpu.VMEM_SHARED((_NP, _C), jnp.float32),
        pltpu.SemaphoreType.DMA,
        pltpu.SemaphoreType.DMA,
        pltpu.SemaphoreType.DMA,
        pltpu.SemaphoreType.DMA,
    ],
)
def _spmm_kernel(xs_hbm, zeros_hbm, src_hbm, dst_hbm, out_hbm,
                 src_v, dst_v, rows_a, rows_b, acc,
                 gsem_a, gsem_b, ssem_a, ssem_b):
    c = lax.axis_index("c")
    s = lax.axis_index("s")
    wid = c * 16 + s
    r0 = s * _RPT

    @pl.when(c == 0)
    def _():
        pltpu.sync_copy(xs_hbm.at[pl.ds(r0, _RPT)], acc.at[pl.ds(r0, _RPT)])

    @pl.when(c == 1)
    def _():
        pltpu.sync_copy(zeros_hbm.at[pl.ds(r0, _RPT)], acc.at[pl.ds(r0, _RPT)])

    plsc.subcore_barrier()

    for h in range(_NPH):
        pltpu.sync_copy(src_hbm.at[wid, h], src_v)
        pltpu.sync_copy(dst_hbm.at[wid, h], dst_v)
        pltpu.async_copy(xs_hbm.at[src_v.at[0]], rows_a, gsem_a)
        pltpu.async_copy(xs_hbm.at[src_v.at[1]], rows_b, gsem_b)

        def body(g2, carry):
            g = g2 * 2
            pltpu.make_async_copy(
                xs_hbm.at[src_v.at[g]], rows_a, gsem_a).wait()
            pltpu.async_copy(rows_a, acc.at[dst_v.at[g]], ssem_a, add=True)
            pltpu.make_async_copy(
                xs_hbm.at[src_v.at[g + 1]], rows_b, gsem_b).wait()
            pltpu.async_copy(rows_b, acc.at[dst_v.at[g + 1]], ssem_b,
                             add=True)
            pltpu.make_async_copy(rows_a, acc.at[dst_v.at[g]],
                                  ssem_a).wait()
            pltpu.async_copy(xs_hbm.at[src_v.at[g + 2]], rows_a, gsem_a)
            pltpu.make_async_copy(rows_b, acc.at[dst_v.at[g + 1]],
                                  ssem_b).wait()
            pltpu.async_copy(xs_hbm.at[src_v.at[g + 3]], rows_b, gsem_b)
            return carry

        lax.fori_loop(0, _NCHH // 2, body, 0)
        pltpu.make_async_copy(xs_hbm.at[src_v.at[0]], rows_a, gsem_a).wait()
        pltpu.make_async_copy(xs_hbm.at[src_v.at[1]], rows_b, gsem_b).wait()

    plsc.subcore_barrier()
    pltpu.sync_copy(acc.at[pl.ds(r0, _RPT)], out_hbm.at[c, pl.ds(r0, _RPT)])


def _pre_body(deg_ref, x_ref, dinv_ref, xs_ref):
    d = deg_ref[:, 0] + deg_ref[:, 1]
    dinv = jnp.where(d > 0, lax.rsqrt(jnp.maximum(d, 1e-12)), 0.0)
    dinv_ref[:, 0] = dinv
    xs_ref[...] = x_ref[...] * dinv[:, None]


def _layer_body(wb_ref, s_ref, dinv_ref, x_ref, xs_ref):
    w = wb_ref[0]
    b = wb_ref[1]
    dinv = dinv_ref[:, 0]
    x = jnp.maximum((s_ref[0] + s_ref[1]) * dinv[:, None] * w + b, 0.0)
    x_ref[...] = x
    xs_ref[...] = x * dinv[:, None]


def _final_body(wb_ref, s_ref, dinv_ref, ro_ref, x0_ref, x1_ref, x2_ref,
                x3_ref, out3_ref, xall_ref):
    w = wb_ref[0]
    b = wb_ref[1]
    dinv = dinv_ref[:, 0]
    x = jnp.maximum((s_ref[0] + s_ref[1]) * dinv[:, None] * w + b, 0.0)
    out3_ref[...] = (x[None, :, :] * ro_ref[0][:, None, None]
                     + ro_ref[1][:, None, None])
    xall_ref[:, 0, :] = x0_ref[...]
    xall_ref[:, 1, :] = x1_ref[...]
    xall_ref[:, 2, :] = x2_ref[...]
    xall_ref[:, 3, :] = x3_ref[...]
    xall_ref[:, 4, :] = x


def kernel(x, edge_index, conv_W, conv_b, ro_W, ro_b):
    f32 = jnp.float32
    pad = (jnp.arange(_TOTAL - _E, dtype=jnp.int32) % _NPAD_ROWS) + _N
    src_flat = jnp.concatenate([edge_index[0], pad])
    dst_flat = jnp.concatenate([edge_index[1], pad])
    src_p = src_flat.reshape(_NT, _NPH, _NCHH, _K)
    dst_p = dst_flat.reshape(_NT, _NPH, _NCHH, _K)
    ring_pad = ((jnp.arange(_NT * _NPH * 2 * _K, dtype=jnp.int32)
                 % _NPAD_ROWS) + _N).reshape(_NT, _NPH, 2, _K)
    src_p = jnp.concatenate([src_p, ring_pad], axis=2)
    dst_deg = dst_flat.reshape(_NT, _NCHD, _KD)
    x_pad = jnp.zeros((_NP, _C), f32).at[:_N].set(x)
    zeros_big = jnp.zeros((_NP, _C), f32)
    ones_np = jnp.ones((_NP,), f32)
    deg_init = jnp.stack(
        [jnp.concatenate([jnp.ones((_N,), f32), jnp.zeros((_NPAD_ROWS,), f32)]),
         jnp.zeros((_NP,), f32)])
    wb = jnp.stack([conv_W[0, 0], conv_b[0]])
    ro = jnp.concatenate([ro_W, ro_b[None, :]], axis=0)

    deg2 = _deg_kernel(dst_deg, deg_init, ones_np)
    deg_t = deg2.T

    grid = (_NP // _BN,)
    dinv, xs = pl.pallas_call(
        _pre_body,
        grid=grid,
        in_specs=[
            pl.BlockSpec((_BN, 2), lambda i: (i, 0)),
            pl.BlockSpec((_BN, _C), lambda i: (i, 0)),
        ],
        out_specs=[
            pl.BlockSpec((_BN, 1), lambda i: (i, 0)),
            pl.BlockSpec((_BN, _C), lambda i: (i, 0)),
        ],
        out_shape=[
            jax.ShapeDtypeStruct((_NP, 1), f32),
            jax.ShapeDtypeStruct((_NP, _C), f32),
        ],
    )(deg_t, x_pad)

    layer_call = pl.pallas_call(
        _layer_body,
        grid=grid,
        in_specs=[
            pl.BlockSpec(memory_space=pltpu.SMEM),
            pl.BlockSpec((2, _BN, _C), lambda i: (0, i, 0)),
            pl.BlockSpec((_BN, 1), lambda i: (i, 0)),
        ],
        out_specs=[
            pl.BlockSpec((_BN, _C), lambda i: (i, 0)),
            pl.BlockSpec((_BN, _C), lambda i: (i, 0)),
        ],
        out_shape=[
            jax.ShapeDtypeStruct((_NP, _C), f32),
            jax.ShapeDtypeStruct((_NP, _C), f32),
        ],
    )

    xs_list = []
    for _ in range(_NLAYERS - 1):
        s_part = _spmm_kernel(xs, zeros_big, src_p, dst_p)
        xlayer, xs = layer_call(wb, s_part, dinv)
        xs_list.append(xlayer)

    s_part = _spmm_kernel(xs, zeros_big, src_p, dst_p)
    out3, x_all = pl.pallas_call(
        _final_body,
        grid=grid,
        in_specs=[
            pl.BlockSpec(memory_space=pltpu.SMEM),
            pl.BlockSpec((2, _BN, _C), lambda i: (0, i, 0)),
            pl.BlockSpec((_BN, 1), lambda i: (i, 0)),
            pl.BlockSpec((2, _NCLASS), lambda i: (0, 0)),
            pl.BlockSpec((_BN, _C), lambda i: (i, 0)),
            pl.BlockSpec((_BN, _C), lambda i: (i, 0)),
            pl.BlockSpec((_BN, _C), lambda i: (i, 0)),
            pl.BlockSpec((_BN, _C), lambda i: (i, 0)),
        ],
        out_specs=[
            pl.BlockSpec((_NCLASS, _BN, _C), lambda i: (0, i, 0)),
            pl.BlockSpec((_BN, _NLAYERS + 1, _C), lambda i: (i, 0, 0)),
        ],
        out_shape=[
            jax.ShapeDtypeStruct((_NCLASS, _NP, _C), f32),
            jax.ShapeDtypeStruct((_N, _NLAYERS + 1, _C), f32),
        ],
    )(wb, s_part, dinv, ro, x_pad, xs_list[0], xs_list[1], xs_list[2])

    out = jnp.transpose(out3, (1, 2, 0))[:_N]
    return (out, x_all)


# speedup vs baseline: 1.6710x; 1.0679x over previous
import functools

import jax
import jax.numpy as jnp
from jax import lax
from jax.experimental import pallas as pl
from jax.experimental.pallas import tpu as pltpu
from jax.experimental.pallas import tpu_sc as plsc

_N = 10000
_C = 128
_NCLASS = 7
_NLAYERS = 4
_E = 320000

_NP = 10240
_NPAD_ROWS = _NP - _N
_NT = 32
_K = 64
_NPH = 2
_NCHH = 80
_TOTAL = _NT * _NPH * _NCHH * _K
_KD = 128
_NCHD = 80
_RPT = _NP // 16
_BN = 1024

_mesh = plsc.VectorSubcoreMesh(core_axis_name="c", subcore_axis_name="s")


@functools.partial(
    pl.kernel,
    out_type=jax.ShapeDtypeStruct((2, _NP), jnp.float32),
    mesh=_mesh,
    scratch_types=[
        pltpu.VMEM((_NCHD, _KD), jnp.int32),
        pltpu.VMEM((_KD,), jnp.float32),
        pltpu.VMEM_SHARED((_NP,), jnp.float32),
    ],
)
def _deg_kernel(dst_hbm, init_hbm, ones_hbm, deg_out, dst_v, ones_v, acc):
    c = lax.axis_index("c")
    s = lax.axis_index("s")
    wid = c * 16 + s
    r0 = s * _RPT
    pltpu.sync_copy(init_hbm.at[c, pl.ds(r0, _RPT)], acc.at[pl.ds(r0, _RPT)])
    pltpu.sync_copy(ones_hbm.at[pl.ds(0, _KD)], ones_v)
    pltpu.sync_copy(dst_hbm.at[wid], dst_v)
    plsc.subcore_barrier()

    def body(j, carry):
        pltpu.sync_copy(ones_v, acc.at[dst_v.at[j]], add=True)
        return carry

    lax.fori_loop(0, _NCHD, body, 0)
    plsc.subcore_barrier()
    pltpu.sync_copy(acc.at[pl.ds(r0, _RPT)], deg_out.at[c, pl.ds(r0, _RPT)])


@functools.partial(
    pl.kernel,
    out_type=jax.ShapeDtypeStruct((2, _NP, _C), jnp.float32),
    mesh=_mesh,
    scratch_types=[
        pltpu.VMEM((_NCHH + 2, _K), jnp.int32),
        pltpu.VMEM((_NCHH, _K), jnp.int32),
        pltpu.VMEM((_K, _C), jnp.float32),
        pltpu.VMEM((_K, _C), jnp.float32),
        pltpu.VMEM_SHARED((_NP, _C), jnp.float32),
        pltpu.SemaphoreType.DMA,
        pltpu.SemaphoreType.DMA,
        pltpu.SemaphoreType.DMA,
        pltpu.SemaphoreType.DMA,
    ],
)
def _spmm_kernel(xs_hbm, zeros_hbm, src_hbm, dst_hbm, out_hbm,
                 src_v, dst_v, rows_a, rows_b, acc,
                 gsem_a, gsem_b, ssem_a, ssem_b):
    c = lax.axis_index("c")
    s = lax.axis_index("s")
    wid = c * 16 + s
    r0 = s * _RPT

    @pl.when(c == 0)
    def _():
        pltpu.sync_copy(xs_hbm.at[pl.ds(r0, _RPT)], acc.at[pl.ds(r0, _RPT)])

    @pl.when(c == 1)
    def _():
        pltpu.sync_copy(zeros_hbm.at[pl.ds(r0, _RPT)], acc.at[pl.ds(r0, _RPT)])

    plsc.subcore_barrier()

    for h in range(_NPH):
        pltpu.sync_copy(src_hbm.at[wid, h], src_v)
        pltpu.sync_copy(dst_hbm.at[wid, h], dst_v)
        pltpu.async_copy(xs_hbm.at[src_v.at[0]], rows_a, gsem_a)
        pltpu.async_copy(xs_hbm.at[src_v.at[1]], rows_b, gsem_b)

        def body(g2, carry):
            g = g2 * 2
            pltpu.make_async_copy(
                xs_hbm.at[src_v.at[g]], rows_a, gsem_a).wait()
            pltpu.async_copy(rows_a, acc.at[dst_v.at[g]], ssem_a, add=True)
            pltpu.make_async_copy(
                xs_hbm.at[src_v.at[g + 1]], rows_b, gsem_b).wait()
            pltpu.async_copy(rows_b, acc.at[dst_v.at[g + 1]], ssem_b,
                             add=True)
            pltpu.make_async_copy(rows_a, acc.at[dst_v.at[g]],
                                  ssem_a).wait()
            pltpu.async_copy(xs_hbm.at[src_v.at[g + 2]], rows_a, gsem_a)
            pltpu.make_async_copy(rows_b, acc.at[dst_v.at[g + 1]],
                                  ssem_b).wait()
            pltpu.async_copy(xs_hbm.at[src_v.at[g + 3]], rows_b, gsem_b)
            return carry

        lax.fori_loop(0, _NCHH // 2, body, 0)
        pltpu.make_async_copy(xs_hbm.at[src_v.at[0]], rows_a, gsem_a).wait()
        pltpu.make_async_copy(xs_hbm.at[src_v.at[1]], rows_b, gsem_b).wait()

    plsc.subcore_barrier()
    pltpu.sync_copy(acc.at[pl.ds(r0, _RPT)], out_hbm.at[c, pl.ds(r0, _RPT)])


def _pre_body(deg_ref, x_ref, dinv_ref, xs_ref):
    d = deg_ref[:, 0] + deg_ref[:, 1]
    dinv = jnp.where(d > 0, lax.rsqrt(jnp.maximum(d, 1e-12)), 0.0)
    dinv_ref[:, 0] = dinv
    xs_ref[...] = x_ref[...] * dinv[:, None]


def _layer_body(wb_ref, s_ref, dinv_ref, x_ref, xs_ref):
    w = wb_ref[0]
    b = wb_ref[1]
    dinv = dinv_ref[:, 0]
    x = jnp.maximum((s_ref[0] + s_ref[1]) * dinv[:, None] * w + b, 0.0)
    x_ref[...] = x
    xs_ref[...] = x * dinv[:, None]


def _final_body(wb_ref, s_ref, dinv_ref, ro_ref, x_ref, out3_ref):
    w = wb_ref[0]
    b = wb_ref[1]
    dinv = dinv_ref[:, 0]
    x = jnp.maximum((s_ref[0] + s_ref[1]) * dinv[:, None] * w + b, 0.0)
    x_ref[...] = x
    out3_ref[...] = (x[None, :, :] * ro_ref[0][:, None, None]
                     + ro_ref[1][:, None, None])


def kernel(x, edge_index, conv_W, conv_b, ro_W, ro_b):
    f32 = jnp.float32
    pad = (jnp.arange(_TOTAL - _E, dtype=jnp.int32) % _NPAD_ROWS) + _N
    src_flat = jnp.concatenate([edge_index[0], pad])
    dst_flat = jnp.concatenate([edge_index[1], pad])
    src_p = src_flat.reshape(_NT, _NPH, _NCHH, _K)
    dst_p = dst_flat.reshape(_NT, _NPH, _NCHH, _K)
    ring_pad = ((jnp.arange(_NT * _NPH * 2 * _K, dtype=jnp.int32)
                 % _NPAD_ROWS) + _N).reshape(_NT, _NPH, 2, _K)
    src_p = jnp.concatenate([src_p, ring_pad], axis=2)
    dst_deg = dst_flat.reshape(_NT, _NCHD, _KD)
    x_pad = jnp.zeros((_NP, _C), f32).at[:_N].set(x)
    zeros_big = jnp.zeros((_NP, _C), f32)
    ones_np = jnp.ones((_NP,), f32)
    deg_init = jnp.stack(
        [jnp.concatenate([jnp.ones((_N,), f32), jnp.zeros((_NPAD_ROWS,), f32)]),
         jnp.zeros((_NP,), f32)])
    wb = jnp.stack([conv_W[0, 0], conv_b[0]])
    ro = jnp.concatenate([ro_W, ro_b[None, :]], axis=0)

    deg2 = _deg_kernel(dst_deg, deg_init, ones_np)
    deg_t = deg2.T

    grid = (_NP // _BN,)
    dinv, xs = pl.pallas_call(
        _pre_body,
        grid=grid,
        in_specs=[
            pl.BlockSpec((_BN, 2), lambda i: (i, 0)),
            pl.BlockSpec((_BN, _C), lambda i: (i, 0)),
        ],
        out_specs=[
            pl.BlockSpec((_BN, 1), lambda i: (i, 0)),
            pl.BlockSpec((_BN, _C), lambda i: (i, 0)),
        ],
        out_shape=[
            jax.ShapeDtypeStruct((_NP, 1), f32),
            jax.ShapeDtypeStruct((_NP, _C), f32),
        ],
    )(deg_t, x_pad)

    layer_call = pl.pallas_call(
        _layer_body,
        grid=grid,
        in_specs=[
            pl.BlockSpec(memory_space=pltpu.SMEM),
            pl.BlockSpec((2, _BN, _C), lambda i: (0, i, 0)),
            pl.BlockSpec((_BN, 1), lambda i: (i, 0)),
        ],
        out_specs=[
            pl.BlockSpec((_BN, _C), lambda i: (i, 0)),
            pl.BlockSpec((_BN, _C), lambda i: (i, 0)),
        ],
        out_shape=[
            jax.ShapeDtypeStruct((_NP, _C), f32),
            jax.ShapeDtypeStruct((_NP, _C), f32),
        ],
    )

    xs_list = []
    for _ in range(_NLAYERS - 1):
        s_part = _spmm_kernel(xs, zeros_big, src_p, dst_p)
        xlayer, xs = layer_call(wb, s_part, dinv)
        xs_list.append(xlayer)

    s_part = _spmm_kernel(xs, zeros_big, src_p, dst_p)
    x_last, out3 = pl.pallas_call(
        _final_body,
        grid=grid,
        in_specs=[
            pl.BlockSpec(memory_space=pltpu.SMEM),
            pl.BlockSpec((2, _BN, _C), lambda i: (0, i, 0)),
            pl.BlockSpec((_BN, 1), lambda i: (i, 0)),
            pl.BlockSpec((2, _NCLASS), lambda i: (0, 0)),
        ],
        out_specs=[
            pl.BlockSpec((_BN, _C), lambda i: (i, 0)),
            pl.BlockSpec((_NCLASS, _BN, _C), lambda i: (0, i, 0)),
        ],
        out_shape=[
            jax.ShapeDtypeStruct((_NP, _C), f32),
            jax.ShapeDtypeStruct((_NCLASS, _NP, _C), f32),
        ],
    )(wb, s_part, dinv, ro)
    xs_list.append(x_last)

    out = jnp.transpose(out3, (1, 2, 0))[:_N]
    x_all = jnp.stack([x] + [xl[:_N] for xl in xs_list], axis=1)
    return (out, x_all)


# speedup vs baseline: 1.6887x; 1.0106x over previous
import functools

import jax
import jax.numpy as jnp
from jax import lax
from jax.experimental import pallas as pl
from jax.experimental.pallas import tpu as pltpu
from jax.experimental.pallas import tpu_sc as plsc

_N = 10000
_C = 128
_NCLASS = 7
_NLAYERS = 4
_E = 320000

_NP = 10240
_NPAD_ROWS = _NP - _N
_NT = 32
_K = 64
_NPH = 2
_NCHH = 80
_TOTAL = _NT * _NPH * _NCHH * _K
_KD = 128
_NCHD = 80
_RPT = _NP // 16
_BN = 2048

_mesh = plsc.VectorSubcoreMesh(core_axis_name="c", subcore_axis_name="s")


@functools.partial(
    pl.kernel,
    out_type=jax.ShapeDtypeStruct((2, _NP), jnp.float32),
    mesh=_mesh,
    scratch_types=[
        pltpu.VMEM((_NCHD, _KD), jnp.int32),
        pltpu.VMEM((_KD,), jnp.float32),
        pltpu.VMEM_SHARED((_NP,), jnp.float32),
    ],
)
def _deg_kernel(dst_hbm, init_hbm, ones_hbm, deg_out, dst_v, ones_v, acc):
    c = lax.axis_index("c")
    s = lax.axis_index("s")
    wid = c * 16 + s
    r0 = s * _RPT
    pltpu.sync_copy(init_hbm.at[c, pl.ds(r0, _RPT)], acc.at[pl.ds(r0, _RPT)])
    pltpu.sync_copy(ones_hbm.at[pl.ds(0, _KD)], ones_v)
    pltpu.sync_copy(dst_hbm.at[wid], dst_v)
    plsc.subcore_barrier()

    def body(j, carry):
        pltpu.sync_copy(ones_v, acc.at[dst_v.at[j]], add=True)
        return carry

    lax.fori_loop(0, _NCHD, body, 0)
    plsc.subcore_barrier()
    pltpu.sync_copy(acc.at[pl.ds(r0, _RPT)], deg_out.at[c, pl.ds(r0, _RPT)])


@functools.partial(
    pl.kernel,
    out_type=jax.ShapeDtypeStruct((2, _NP, _C), jnp.float32),
    mesh=_mesh,
    scratch_types=[
        pltpu.VMEM((_NCHH + 2, _K), jnp.int32),
        pltpu.VMEM((_NCHH, _K), jnp.int32),
        pltpu.VMEM((_K, _C), jnp.float32),
        pltpu.VMEM((_K, _C), jnp.float32),
        pltpu.VMEM_SHARED((_NP, _C), jnp.float32),
        pltpu.SemaphoreType.DMA,
        pltpu.SemaphoreType.DMA,
        pltpu.SemaphoreType.DMA,
        pltpu.SemaphoreType.DMA,
    ],
)
def _spmm_kernel(xs_hbm, zeros_hbm, src_hbm, dst_hbm, out_hbm,
                 src_v, dst_v, rows_a, rows_b, acc,
                 gsem_a, gsem_b, ssem_a, ssem_b):
    c = lax.axis_index("c")
    s = lax.axis_index("s")
    wid = c * 16 + s
    r0 = s * _RPT

    @pl.when(c == 0)
    def _():
        pltpu.sync_copy(xs_hbm.at[pl.ds(r0, _RPT)], acc.at[pl.ds(r0, _RPT)])

    @pl.when(c == 1)
    def _():
        pltpu.sync_copy(zeros_hbm.at[pl.ds(r0, _RPT)], acc.at[pl.ds(r0, _RPT)])

    plsc.subcore_barrier()

    for h in range(_NPH):
        pltpu.sync_copy(src_hbm.at[wid, h], src_v)
        pltpu.sync_copy(dst_hbm.at[wid, h], dst_v)
        pltpu.async_copy(xs_hbm.at[src_v.at[0]], rows_a, gsem_a)
        pltpu.async_copy(xs_hbm.at[src_v.at[1]], rows_b, gsem_b)

        def body(g2, carry):
            g = g2 * 2
            pltpu.make_async_copy(
                xs_hbm.at[src_v.at[g]], rows_a, gsem_a).wait()
            pltpu.async_copy(rows_a, acc.at[dst_v.at[g]], ssem_a, add=True)
            pltpu.make_async_copy(
                xs_hbm.at[src_v.at[g + 1]], rows_b, gsem_b).wait()
            pltpu.async_copy(rows_b, acc.at[dst_v.at[g + 1]], ssem_b,
                             add=True)
            pltpu.make_async_copy(rows_a, acc.at[dst_v.at[g]],
                                  ssem_a).wait()
            pltpu.async_copy(xs_hbm.at[src_v.at[g + 2]], rows_a, gsem_a)
            pltpu.make_async_copy(rows_b, acc.at[dst_v.at[g + 1]],
                                  ssem_b).wait()
            pltpu.async_copy(xs_hbm.at[src_v.at[g + 3]], rows_b, gsem_b)
            return carry

        lax.fori_loop(0, _NCHH // 2, body, 0)
        pltpu.make_async_copy(xs_hbm.at[src_v.at[0]], rows_a, gsem_a).wait()
        pltpu.make_async_copy(xs_hbm.at[src_v.at[1]], rows_b, gsem_b).wait()

    plsc.subcore_barrier()
    pltpu.sync_copy(acc.at[pl.ds(r0, _RPT)], out_hbm.at[c, pl.ds(r0, _RPT)])


def _pre_body(deg_ref, x_ref, dinv_ref, xs_ref):
    d = deg_ref[:, 0] + deg_ref[:, 1]
    dinv = jnp.where(d > 0, lax.rsqrt(jnp.maximum(d, 1e-12)), 0.0)
    dinv_ref[:, 0] = dinv
    xs_ref[...] = x_ref[...] * dinv[:, None]


def _layer_body(wb_ref, s_ref, dinv_ref, x_ref, xs_ref):
    w = wb_ref[0]
    b = wb_ref[1]
    dinv = dinv_ref[:, 0]
    x = jnp.maximum((s_ref[0] + s_ref[1]) * dinv[:, None] * w + b, 0.0)
    x_ref[...] = x
    xs_ref[...] = x * dinv[:, None]


def _final_body(wb_ref, s_ref, dinv_ref, ro_ref, x_ref, out3_ref):
    w = wb_ref[0]
    b = wb_ref[1]
    dinv = dinv_ref[:, 0]
    x = jnp.maximum((s_ref[0] + s_ref[1]) * dinv[:, None] * w + b, 0.0)
    x_ref[...] = x
    out3_ref[...] = (x[None, :, :] * ro_ref[0][:, None, None]
                     + ro_ref[1][:, None, None])


def kernel(x, edge_index, conv_W, conv_b, ro_W, ro_b):
    f32 = jnp.float32
    pad = (jnp.arange(_TOTAL - _E, dtype=jnp.int32) % _NPAD_ROWS) + _N
    src_flat = jnp.concatenate([edge_index[0], pad])
    dst_flat = jnp.concatenate([edge_index[1], pad])
    src_p = src_flat.reshape(_NT, _NPH, _NCHH, _K)
    dst_p = dst_flat.reshape(_NT, _NPH, _NCHH, _K)
    ring_pad = ((jnp.arange(_NT * _NPH * 2 * _K, dtype=jnp.int32)
                 % _NPAD_ROWS) + _N).reshape(_NT, _NPH, 2, _K)
    src_p = jnp.concatenate([src_p, ring_pad], axis=2)
    dst_deg = dst_flat.reshape(_NT, _NCHD, _KD)
    x_pad = jnp.zeros((_NP, _C), f32).at[:_N].set(x)
    zeros_big = jnp.zeros((_NP, _C), f32)
    ones_np = jnp.ones((_NP,), f32)
    deg_init = jnp.stack(
        [jnp.concatenate([jnp.ones((_N,), f32), jnp.zeros((_NPAD_ROWS,), f32)]),
         jnp.zeros((_NP,), f32)])
    wb = jnp.stack([conv_W[0, 0], conv_b[0]])
    ro = jnp.concatenate([ro_W, ro_b[None, :]], axis=0)

    deg2 = _deg_kernel(dst_deg, deg_init, ones_np)
    deg_t = deg2.T

    grid = (_NP // _BN,)
    dinv, xs = pl.pallas_call(
        _pre_body,
        grid=grid,
        in_specs=[
            pl.BlockSpec((_BN, 2), lambda i: (i, 0)),
            pl.BlockSpec((_BN, _C), lambda i: (i, 0)),
        ],
        out_specs=[
            pl.BlockSpec((_BN, 1), lambda i: (i, 0)),
            pl.BlockSpec((_BN, _C), lambda i: (i, 0)),
        ],
        out_shape=[
            jax.ShapeDtypeStruct((_NP, 1), f32),
            jax.ShapeDtypeStruct((_NP, _C), f32),
        ],
    )(deg_t, x_pad)

    layer_call = pl.pallas_call(
        _layer_body,
        grid=grid,
        in_specs=[
            pl.BlockSpec(memory_space=pltpu.SMEM),
            pl.BlockSpec((2, _BN, _C), lambda i: (0, i, 0)),
            pl.BlockSpec((_BN, 1), lambda i: (i, 0)),
        ],
        out_specs=[
            pl.BlockSpec((_BN, _C), lambda i: (i, 0)),
            pl.BlockSpec((_BN, _C), lambda i: (i, 0)),
        ],
        out_shape=[
            jax.ShapeDtypeStruct((_NP, _C), f32),
            jax.ShapeDtypeStruct((_NP, _C), f32),
        ],
    )

    xs_list = []
    for _ in range(_NLAYERS - 1):
        s_part = _spmm_kernel(xs, zeros_big, src_p, dst_p)
        xlayer, xs = layer_call(wb, s_part, dinv)
        xs_list.append(xlayer)

    s_part = _spmm_kernel(xs, zeros_big, src_p, dst_p)
    x_last, out3 = pl.pallas_call(
        _final_body,
        grid=grid,
        in_specs=[
            pl.BlockSpec(memory_space=pltpu.SMEM),
            pl.BlockSpec((2, _BN, _C), lambda i: (0, i, 0)),
            pl.BlockSpec((_BN, 1), lambda i: (i, 0)),
            pl.BlockSpec((2, _NCLASS), lambda i: (0, 0)),
        ],
        out_specs=[
            pl.BlockSpec((_BN, _C), lambda i: (i, 0)),
            pl.BlockSpec((_NCLASS, _BN, _C), lambda i: (0, i, 0)),
        ],
        out_shape=[
            jax.ShapeDtypeStruct((_NP, _C), f32),
            jax.ShapeDtypeStruct((_NCLASS, _NP, _C), f32),
        ],
    )(wb, s_part, dinv, ro)
    xs_list.append(x_last)

    out = jnp.transpose(out3, (1, 2, 0))[:_N]
    x_all = jnp.stack([x] + [xl[:_N] for xl in xs_list], axis=1)
    return (out, x_all)


# speedup vs baseline: 1.6949x; 1.0037x over previous
import functools

import jax
import jax.numpy as jnp
from jax import lax
from jax.experimental import pallas as pl
from jax.experimental.pallas import tpu as pltpu
from jax.experimental.pallas import tpu_sc as plsc

_N = 10000
_C = 128
_NCLASS = 7
_NLAYERS = 4
_E = 320000

_NP = 10240
_NPAD_ROWS = _NP - _N
_NT = 32
_K = 64
_NPH = 2
_NCHH = 80
_TOTAL = _NT * _NPH * _NCHH * _K
_KD = 128
_NCHD = 80
_RPT = _NP // 16
_BN = 2560

_mesh = plsc.VectorSubcoreMesh(core_axis_name="c", subcore_axis_name="s")


@functools.partial(
    pl.kernel,
    out_type=jax.ShapeDtypeStruct((2, _NP), jnp.float32),
    mesh=_mesh,
    scratch_types=[
        pltpu.VMEM((_NCHD, _KD), jnp.int32),
        pltpu.VMEM((_KD,), jnp.float32),
        pltpu.VMEM_SHARED((_NP,), jnp.float32),
    ],
)
def _deg_kernel(dst_hbm, init_hbm, ones_hbm, deg_out, dst_v, ones_v, acc):
    c = lax.axis_index("c")
    s = lax.axis_index("s")
    wid = c * 16 + s
    r0 = s * _RPT
    pltpu.sync_copy(init_hbm.at[c, pl.ds(r0, _RPT)], acc.at[pl.ds(r0, _RPT)])
    pltpu.sync_copy(ones_hbm.at[pl.ds(0, _KD)], ones_v)
    pltpu.sync_copy(dst_hbm.at[wid], dst_v)
    plsc.subcore_barrier()

    def body(j, carry):
        pltpu.sync_copy(ones_v, acc.at[dst_v.at[j]], add=True)
        return carry

    lax.fori_loop(0, _NCHD, body, 0)
    plsc.subcore_barrier()
    pltpu.sync_copy(acc.at[pl.ds(r0, _RPT)], deg_out.at[c, pl.ds(r0, _RPT)])


@functools.partial(
    pl.kernel,
    out_type=jax.ShapeDtypeStruct((2, _NP, _C), jnp.float32),
    mesh=_mesh,
    scratch_types=[
        pltpu.VMEM((_NCHH + 2, _K), jnp.int32),
        pltpu.VMEM((_NCHH, _K), jnp.int32),
        pltpu.VMEM((_K, _C), jnp.float32),
        pltpu.VMEM((_K, _C), jnp.float32),
        pltpu.VMEM_SHARED((_NP, _C), jnp.float32),
        pltpu.SemaphoreType.DMA,
        pltpu.SemaphoreType.DMA,
        pltpu.SemaphoreType.DMA,
        pltpu.SemaphoreType.DMA,
    ],
)
def _spmm_kernel(xs_hbm, zeros_hbm, src_hbm, dst_hbm, out_hbm,
                 src_v, dst_v, rows_a, rows_b, acc,
                 gsem_a, gsem_b, ssem_a, ssem_b):
    c = lax.axis_index("c")
    s = lax.axis_index("s")
    wid = c * 16 + s
    r0 = s * _RPT

    @pl.when(c == 0)
    def _():
        pltpu.sync_copy(xs_hbm.at[pl.ds(r0, _RPT)], acc.at[pl.ds(r0, _RPT)])

    @pl.when(c == 1)
    def _():
        pltpu.sync_copy(zeros_hbm.at[pl.ds(r0, _RPT)], acc.at[pl.ds(r0, _RPT)])

    plsc.subcore_barrier()

    for h in range(_NPH):
        pltpu.sync_copy(src_hbm.at[wid, h], src_v)
        pltpu.sync_copy(dst_hbm.at[wid, h], dst_v)
        pltpu.async_copy(xs_hbm.at[src_v.at[0]], rows_a, gsem_a)
        pltpu.async_copy(xs_hbm.at[src_v.at[1]], rows_b, gsem_b)

        def body(g2, carry):
            g = g2 * 2
            pltpu.make_async_copy(
                xs_hbm.at[src_v.at[g]], rows_a, gsem_a).wait()
            pltpu.async_copy(rows_a, acc.at[dst_v.at[g]], ssem_a, add=True)
            pltpu.make_async_copy(
                xs_hbm.at[src_v.at[g + 1]], rows_b, gsem_b).wait()
            pltpu.async_copy(rows_b, acc.at[dst_v.at[g + 1]], ssem_b,
                             add=True)
            pltpu.make_async_copy(rows_a, acc.at[dst_v.at[g]],
                                  ssem_a).wait()
            pltpu.async_copy(xs_hbm.at[src_v.at[g + 2]], rows_a, gsem_a)
            pltpu.make_async_copy(rows_b, acc.at[dst_v.at[g + 1]],
                                  ssem_b).wait()
            pltpu.async_copy(xs_hbm.at[src_v.at[g + 3]], rows_b, gsem_b)
            return carry

        lax.fori_loop(0, _NCHH // 2, body, 0)
        pltpu.make_async_copy(xs_hbm.at[src_v.at[0]], rows_a, gsem_a).wait()
        pltpu.make_async_copy(xs_hbm.at[src_v.at[1]], rows_b, gsem_b).wait()

    plsc.subcore_barrier()
    pltpu.sync_copy(acc.at[pl.ds(r0, _RPT)], out_hbm.at[c, pl.ds(r0, _RPT)])


def _pre_body(deg_ref, x_ref, dinv_ref, xs_ref):
    d = deg_ref[:, 0] + deg_ref[:, 1]
    dinv = jnp.where(d > 0, lax.rsqrt(jnp.maximum(d, 1e-12)), 0.0)
    dinv_ref[:, 0] = dinv
    xs_ref[...] = x_ref[...] * dinv[:, None]


def _layer_body(wb_ref, s_ref, dinv_ref, x_ref, xs_ref):
    w = wb_ref[0]
    b = wb_ref[1]
    dinv = dinv_ref[:, 0]
    x = jnp.maximum((s_ref[0] + s_ref[1]) * dinv[:, None] * w + b, 0.0)
    x_ref[...] = x
    xs_ref[...] = x * dinv[:, None]


def _final_body(wb_ref, s_ref, dinv_ref, ro_ref, x_ref, out3_ref):
    w = wb_ref[0]
    b = wb_ref[1]
    dinv = dinv_ref[:, 0]
    x = jnp.maximum((s_ref[0] + s_ref[1]) * dinv[:, None] * w + b, 0.0)
    x_ref[...] = x
    out3_ref[...] = (x[None, :, :] * ro_ref[0][:, None, None]
                     + ro_ref[1][:, None, None])


def kernel(x, edge_index, conv_W, conv_b, ro_W, ro_b):
    f32 = jnp.float32
    pad = (jnp.arange(_TOTAL - _E, dtype=jnp.int32) % _NPAD_ROWS) + _N
    src_flat = jnp.concatenate([edge_index[0], pad])
    dst_flat = jnp.concatenate([edge_index[1], pad])
    src_p = src_flat.reshape(_NT, _NPH, _NCHH, _K)
    dst_p = dst_flat.reshape(_NT, _NPH, _NCHH, _K)
    ring_pad = ((jnp.arange(_NT * _NPH * 2 * _K, dtype=jnp.int32)
                 % _NPAD_ROWS) + _N).reshape(_NT, _NPH, 2, _K)
    src_p = jnp.concatenate([src_p, ring_pad], axis=2)
    dst_deg = dst_flat.reshape(_NT, _NCHD, _KD)
    x_pad = jnp.zeros((_NP, _C), f32).at[:_N].set(x)
    zeros_big = jnp.zeros((_NP, _C), f32)
    ones_np = jnp.ones((_NP,), f32)
    deg_init = jnp.stack(
        [jnp.concatenate([jnp.ones((_N,), f32), jnp.zeros((_NPAD_ROWS,), f32)]),
         jnp.zeros((_NP,), f32)])
    wb = jnp.stack([conv_W[0, 0], conv_b[0]])
    ro = jnp.concatenate([ro_W, ro_b[None, :]], axis=0)

    deg2 = _deg_kernel(dst_deg, deg_init, ones_np)
    deg_t = deg2.T

    grid = (_NP // _BN,)
    dinv, xs = pl.pallas_call(
        _pre_body,
        grid=grid,
        in_specs=[
            pl.BlockSpec((_BN, 2), lambda i: (i, 0)),
            pl.BlockSpec((_BN, _C), lambda i: (i, 0)),
        ],
        out_specs=[
            pl.BlockSpec((_BN, 1), lambda i: (i, 0)),
            pl.BlockSpec((_BN, _C), lambda i: (i, 0)),
        ],
        out_shape=[
            jax.ShapeDtypeStruct((_NP, 1), f32),
            jax.ShapeDtypeStruct((_NP, _C), f32),
        ],
    )(deg_t, x_pad)

    layer_call = pl.pallas_call(
        _layer_body,
        grid=grid,
        in_specs=[
            pl.BlockSpec(memory_space=pltpu.SMEM),
            pl.BlockSpec((2, _BN, _C), lambda i: (0, i, 0)),
            pl.BlockSpec((_BN, 1), lambda i: (i, 0)),
        ],
        out_specs=[
            pl.BlockSpec((_BN, _C), lambda i: (i, 0)),
            pl.BlockSpec((_BN, _C), lambda i: (i, 0)),
        ],
        out_shape=[
            jax.ShapeDtypeStruct((_NP, _C), f32),
            jax.ShapeDtypeStruct((_NP, _C), f32),
        ],
    )

    xs_list = []
    for _ in range(_NLAYERS - 1):
        s_part = _spmm_kernel(xs, zeros_big, src_p, dst_p)
        xlayer, xs = layer_call(wb, s_part, dinv)
        xs_list.append(xlayer)

    s_part = _spmm_kernel(xs, zeros_big, src_p, dst_p)
    x_last, out3 = pl.pallas_call(
        _final_body,
        grid=grid,
        in_specs=[
            pl.BlockSpec(memory_space=pltpu.SMEM),
            pl.BlockSpec((2, _BN, _C), lambda i: (0, i, 0)),
            pl.BlockSpec((_BN, 1), lambda i: (i, 0)),
            pl.BlockSpec((2, _NCLASS), lambda i: (0, 0)),
        ],
        out_specs=[
            pl.BlockSpec((_BN, _C), lambda i: (i, 0)),
            pl.BlockSpec((_NCLASS, _BN, _C), lambda i: (0, i, 0)),
        ],
        out_shape=[
            jax.ShapeDtypeStruct((_NP, _C), f32),
            jax.ShapeDtypeStruct((_NCLASS, _NP, _C), f32),
        ],
    )(wb, s_part, dinv, ro)
    xs_list.append(x_last)

    out = jnp.transpose(out3, (1, 2, 0))[:_N]
    x_all = jnp.stack([x] + [xl[:_N] for xl in xs_list], axis=1)
    return (out, x_all)


# speedup vs baseline: 1.7079x; 1.0077x over previous
import functools

import jax
import jax.numpy as jnp
from jax import lax
from jax.experimental import pallas as pl
from jax.experimental.pallas import tpu as pltpu
from jax.experimental.pallas import tpu_sc as plsc

_N = 10000
_C = 128
_NCLASS = 7
_NLAYERS = 4
_E = 320000

_NP = 10240
_NPAD_ROWS = _NP - _N
_NT = 32
_K = 64
_NPH = 2
_NCHH = 80
_TOTAL = _NT * _NPH * _NCHH * _K
_KD = 128
_NCHD = 80
_RPT = _NP // 16
_BN = 2560

_mesh = plsc.VectorSubcoreMesh(core_axis_name="c", subcore_axis_name="s")


@functools.partial(
    pl.kernel,
    out_type=jax.ShapeDtypeStruct((2, _NP), jnp.float32),
    mesh=_mesh,
    scratch_types=[
        pltpu.VMEM((_NCHD, _KD), jnp.int32),
        pltpu.VMEM((_KD,), jnp.float32),
        pltpu.VMEM_SHARED((_NP,), jnp.float32),
        pltpu.SemaphoreType.DMA,
    ],
)
def _deg_kernel(dst_hbm, init_hbm, ones_hbm, deg_out, dst_v, ones_v, acc,
                sem):
    c = lax.axis_index("c")
    s = lax.axis_index("s")
    wid = c * 16 + s
    r0 = s * _RPT
    pltpu.sync_copy(init_hbm.at[c, pl.ds(r0, _RPT)], acc.at[pl.ds(r0, _RPT)])
    pltpu.sync_copy(ones_hbm.at[pl.ds(0, _KD)], ones_v)
    pltpu.sync_copy(dst_hbm.at[wid], dst_v)
    plsc.subcore_barrier()

    def body(j, carry):
        pltpu.async_copy(ones_v, acc.at[dst_v.at[j]], sem, add=True)

        @pl.when(j >= 8)
        def _():
            pltpu.make_async_copy(ones_v, acc.at[dst_v.at[0]], sem).wait()

        return carry

    lax.fori_loop(0, _NCHD, body, 0)

    def drain(j, carry):
        pltpu.make_async_copy(ones_v, acc.at[dst_v.at[0]], sem).wait()
        return carry

    lax.fori_loop(0, 8, drain, 0)
    plsc.subcore_barrier()
    pltpu.sync_copy(acc.at[pl.ds(r0, _RPT)], deg_out.at[c, pl.ds(r0, _RPT)])


@functools.partial(
    pl.kernel,
    out_type=jax.ShapeDtypeStruct((2, _NP, _C), jnp.float32),
    mesh=_mesh,
    scratch_types=[
        pltpu.VMEM((_NCHH + 2, _K), jnp.int32),
        pltpu.VMEM((_NCHH, _K), jnp.int32),
        pltpu.VMEM((_K, _C), jnp.float32),
        pltpu.VMEM((_K, _C), jnp.float32),
        pltpu.VMEM_SHARED((_NP, _C), jnp.float32),
        pltpu.SemaphoreType.DMA,
        pltpu.SemaphoreType.DMA,
        pltpu.SemaphoreType.DMA,
        pltpu.SemaphoreType.DMA,
    ],
)
def _spmm_kernel(xs_hbm, zeros_hbm, src_hbm, dst_hbm, out_hbm,
                 src_v, dst_v, rows_a, rows_b, acc,
                 gsem_a, gsem_b, ssem_a, ssem_b):
    c = lax.axis_index("c")
    s = lax.axis_index("s")
    wid = c * 16 + s
    r0 = s * _RPT

    @pl.when(c == 0)
    def _():
        pltpu.sync_copy(xs_hbm.at[pl.ds(r0, _RPT)], acc.at[pl.ds(r0, _RPT)])

    @pl.when(c == 1)
    def _():
        pltpu.sync_copy(zeros_hbm.at[pl.ds(r0, _RPT)], acc.at[pl.ds(r0, _RPT)])

    plsc.subcore_barrier()

    for h in range(_NPH):
        pltpu.sync_copy(src_hbm.at[wid, h], src_v)
        pltpu.sync_copy(dst_hbm.at[wid, h], dst_v)
        pltpu.async_copy(xs_hbm.at[src_v.at[0]], rows_a, gsem_a)
        pltpu.async_copy(xs_hbm.at[src_v.at[1]], rows_b, gsem_b)

        def body(g2, carry):
            g = g2 * 2
            pltpu.make_async_copy(
                xs_hbm.at[src_v.at[g]], rows_a, gsem_a).wait()
            pltpu.async_copy(rows_a, acc.at[dst_v.at[g]], ssem_a, add=True)
            pltpu.make_async_copy(
                xs_hbm.at[src_v.at[g + 1]], rows_b, gsem_b).wait()
            pltpu.async_copy(rows_b, acc.at[dst_v.at[g + 1]], ssem_b,
                             add=True)
            pltpu.make_async_copy(rows_a, acc.at[dst_v.at[g]],
                                  ssem_a).wait()
            pltpu.async_copy(xs_hbm.at[src_v.at[g + 2]], rows_a, gsem_a)
            pltpu.make_async_copy(rows_b, acc.at[dst_v.at[g + 1]],
                                  ssem_b).wait()
            pltpu.async_copy(xs_hbm.at[src_v.at[g + 3]], rows_b, gsem_b)
            return carry

        lax.fori_loop(0, _NCHH // 2, body, 0)
        pltpu.make_async_copy(xs_hbm.at[src_v.at[0]], rows_a, gsem_a).wait()
        pltpu.make_async_copy(xs_hbm.at[src_v.at[1]], rows_b, gsem_b).wait()

    plsc.subcore_barrier()
    pltpu.sync_copy(acc.at[pl.ds(r0, _RPT)], out_hbm.at[c, pl.ds(r0, _RPT)])


def _pre_body(deg_ref, x_ref, dinv_ref, xs_ref):
    d = deg_ref[:, 0] + deg_ref[:, 1]
    dinv = jnp.where(d > 0, lax.rsqrt(jnp.maximum(d, 1e-12)), 0.0)
    dinv_ref[:, 0] = dinv
    xs_ref[...] = x_ref[...] * dinv[:, None]


def _layer_body(wb_ref, s_ref, dinv_ref, x_ref, xs_ref):
    w = wb_ref[0]
    b = wb_ref[1]
    dinv = dinv_ref[:, 0]
    x = jnp.maximum((s_ref[0] + s_ref[1]) * dinv[:, None] * w + b, 0.0)
    x_ref[...] = x
    xs_ref[...] = x * dinv[:, None]


def _final_body(wb_ref, s_ref, dinv_ref, ro_ref, x_ref, out3_ref):
    w = wb_ref[0]
    b = wb_ref[1]
    dinv = dinv_ref[:, 0]
    x = jnp.maximum((s_ref[0] + s_ref[1]) * dinv[:, None] * w + b, 0.0)
    x_ref[...] = x
    out3_ref[...] = (x[None, :, :] * ro_ref[0][:, None, None]
                     + ro_ref[1][:, None, None])


def kernel(x, edge_index, conv_W, conv_b, ro_W, ro_b):
    f32 = jnp.float32
    pad = (jnp.arange(_TOTAL - _E, dtype=jnp.int32) % _NPAD_ROWS) + _N
    src_flat = jnp.concatenate([edge_index[0], pad])
    dst_flat = jnp.concatenate([edge_index[1], pad])
    src_p = src_flat.reshape(_NT, _NPH, _NCHH, _K)
    dst_p = dst_flat.reshape(_NT, _NPH, _NCHH, _K)
    ring_pad = ((jnp.arange(_NT * _NPH * 2 * _K, dtype=jnp.int32)
                 % _NPAD_ROWS) + _N).reshape(_NT, _NPH, 2, _K)
    src_p = jnp.concatenate([src_p, ring_pad], axis=2)
    dst_deg = dst_flat.reshape(_NT, _NCHD, _KD)
    x_pad = jnp.zeros((_NP, _C), f32).at[:_N].set(x)
    zeros_big = jnp.zeros((_NP, _C), f32)
    ones_np = jnp.ones((_NP,), f32)
    deg_init = jnp.stack(
        [jnp.concatenate([jnp.ones((_N,), f32), jnp.zeros((_NPAD_ROWS,), f32)]),
         jnp.zeros((_NP,), f32)])
    wb = jnp.stack([conv_W[0, 0], conv_b[0]])
    ro = jnp.concatenate([ro_W, ro_b[None, :]], axis=0)

    deg2 = _deg_kernel(dst_deg, deg_init, ones_np)
    deg_t = deg2.T

    grid = (_NP // _BN,)
    dinv, xs = pl.pallas_call(
        _pre_body,
        grid=grid,
        in_specs=[
            pl.BlockSpec((_BN, 2), lambda i: (i, 0)),
            pl.BlockSpec((_BN, _C), lambda i: (i, 0)),
        ],
        out_specs=[
            pl.BlockSpec((_BN, 1), lambda i: (i, 0)),
            pl.BlockSpec((_BN, _C), lambda i: (i, 0)),
        ],
        out_shape=[
            jax.ShapeDtypeStruct((_NP, 1), f32),
            jax.ShapeDtypeStruct((_NP, _C), f32),
        ],
    )(deg_t, x_pad)

    layer_call = pl.pallas_call(
        _layer_body,
        grid=grid,
        in_specs=[
            pl.BlockSpec(memory_space=pltpu.SMEM),
            pl.BlockSpec((2, _BN, _C), lambda i: (0, i, 0)),
            pl.BlockSpec((_BN, 1), lambda i: (i, 0)),
        ],
        out_specs=[
            pl.BlockSpec((_BN, _C), lambda i: (i, 0)),
            pl.BlockSpec((_BN, _C), lambda i: (i, 0)),
        ],
        out_shape=[
            jax.ShapeDtypeStruct((_NP, _C), f32),
            jax.ShapeDtypeStruct((_NP, _C), f32),
        ],
    )

    xs_list = []
    for _ in range(_NLAYERS - 1):
        s_part = _spmm_kernel(xs, zeros_big, src_p, dst_p)
        xlayer, xs = layer_call(wb, s_part, dinv)
        xs_list.append(xlayer)

    s_part = _spmm_kernel(xs, zeros_big, src_p, dst_p)
    x_last, out3 = pl.pallas_call(
        _final_body,
        grid=grid,
        in_specs=[
            pl.BlockSpec(memory_space=pltpu.SMEM),
            pl.BlockSpec((2, _BN, _C), lambda i: (0, i, 0)),
            pl.BlockSpec((_BN, 1), lambda i: (i, 0)),
            pl.BlockSpec((2, _NCLASS), lambda i: (0, 0)),
        ],
        out_specs=[
            pl.BlockSpec((_BN, _C), lambda i: (i, 0)),
            pl.BlockSpec((_NCLASS, _BN, _C), lambda i: (0, i, 0)),
        ],
        out_shape=[
            jax.ShapeDtypeStruct((_NP, _C), f32),
            jax.ShapeDtypeStruct((_NCLASS, _NP, _C), f32),
        ],
    )(wb, s_part, dinv, ro)
    xs_list.append(x_last)

    out = jnp.transpose(out3, (1, 2, 0))[:_N]
    x_all = jnp.stack([x] + [xl[:_N] for xl in xs_list], axis=1)
    return (out, x_all)


# speedup vs baseline: 1.7104x; 1.0014x over previous
import functools

import jax
import jax.numpy as jnp
from jax import lax
from jax.experimental import pallas as pl
from jax.experimental.pallas import tpu as pltpu
from jax.experimental.pallas import tpu_sc as plsc

_N = 10000
_C = 128
_NCLASS = 7
_NLAYERS = 4
_E = 320000

_NP = 10240
_NPAD_ROWS = _NP - _N
_NT = 32
_K = 64
_NPH = 2
_NCHH = 80
_TOTAL = _NT * _NPH * _NCHH * _K
_KD = 128
_NCHD = 80
_RPT = _NP // 16
_BN = 2560

_mesh = plsc.VectorSubcoreMesh(core_axis_name="c", subcore_axis_name="s")


@functools.partial(
    pl.kernel,
    out_type=jax.ShapeDtypeStruct((2, _NP), jnp.float32),
    mesh=_mesh,
    scratch_types=[
        pltpu.VMEM((_NCHD, _KD), jnp.int32),
        pltpu.VMEM((_KD,), jnp.float32),
        pltpu.VMEM_SHARED((_NP,), jnp.float32),
        pltpu.SemaphoreType.DMA,
    ],
)
def _deg_kernel(dst_hbm, init_hbm, ones_hbm, deg_out, dst_v, ones_v, acc,
                sem):
    c = lax.axis_index("c")
    s = lax.axis_index("s")
    wid = c * 16 + s
    r0 = s * _RPT
    pltpu.sync_copy(init_hbm.at[c, pl.ds(r0, _RPT)], acc.at[pl.ds(r0, _RPT)])
    pltpu.sync_copy(ones_hbm.at[pl.ds(0, _KD)], ones_v)
    pltpu.sync_copy(dst_hbm.at[wid], dst_v)
    plsc.subcore_barrier()

    def body(j, carry):
        pltpu.async_copy(ones_v, acc.at[dst_v.at[j]], sem, add=True)

        @pl.when(j >= 8)
        def _():
            pltpu.make_async_copy(ones_v, acc.at[dst_v.at[0]], sem).wait()

        return carry

    lax.fori_loop(0, _NCHD, body, 0)

    def drain(j, carry):
        pltpu.make_async_copy(ones_v, acc.at[dst_v.at[0]], sem).wait()
        return carry

    lax.fori_loop(0, 8, drain, 0)
    plsc.subcore_barrier()
    pltpu.sync_copy(acc.at[pl.ds(r0, _RPT)], deg_out.at[c, pl.ds(r0, _RPT)])


@functools.partial(
    pl.kernel,
    out_type=jax.ShapeDtypeStruct((2, _NP, _C), jnp.float32),
    mesh=_mesh,
    scratch_types=[
        pltpu.VMEM((_NCHH + 2, _K), jnp.int32),
        pltpu.VMEM((_NCHH, _K), jnp.int32),
        pltpu.VMEM((_K, _C), jnp.float32),
        pltpu.VMEM((_K, _C), jnp.float32),
        pltpu.VMEM_SHARED((_NP, _C), jnp.float32),
        pltpu.SemaphoreType.DMA,
        pltpu.SemaphoreType.DMA,
        pltpu.SemaphoreType.DMA,
        pltpu.SemaphoreType.DMA,
    ],
)
def _spmm_kernel(xs_hbm, zeros_hbm, src_hbm, dst_hbm, out_hbm,
                 src_v, dst_v, rows_a, rows_b, acc,
                 gsem_a, gsem_b, ssem_a, ssem_b):
    c = lax.axis_index("c")
    s = lax.axis_index("s")
    wid = c * 16 + s
    r0 = s * _RPT

    @pl.when(c == 0)
    def _():
        pltpu.sync_copy(xs_hbm.at[pl.ds(r0, _RPT)], acc.at[pl.ds(r0, _RPT)])

    @pl.when(c == 1)
    def _():
        pltpu.sync_copy(zeros_hbm.at[pl.ds(r0, _RPT)], acc.at[pl.ds(r0, _RPT)])

    for h in range(_NPH):
        pltpu.sync_copy(src_hbm.at[wid, h], src_v)
        pltpu.sync_copy(dst_hbm.at[wid, h], dst_v)
        pltpu.async_copy(xs_hbm.at[src_v.at[0]], rows_a, gsem_a)
        pltpu.async_copy(xs_hbm.at[src_v.at[1]], rows_b, gsem_b)
        if h == 0:
            plsc.subcore_barrier()

        def body(g2, carry):
            g = g2 * 2
            pltpu.make_async_copy(
                xs_hbm.at[src_v.at[g]], rows_a, gsem_a).wait()
            pltpu.async_copy(rows_a, acc.at[dst_v.at[g]], ssem_a, add=True)
            pltpu.make_async_copy(
                xs_hbm.at[src_v.at[g + 1]], rows_b, gsem_b).wait()
            pltpu.async_copy(rows_b, acc.at[dst_v.at[g + 1]], ssem_b,
                             add=True)
            pltpu.make_async_copy(rows_a, acc.at[dst_v.at[g]],
                                  ssem_a).wait()
            pltpu.async_copy(xs_hbm.at[src_v.at[g + 2]], rows_a, gsem_a)
            pltpu.make_async_copy(rows_b, acc.at[dst_v.at[g + 1]],
                                  ssem_b).wait()
            pltpu.async_copy(xs_hbm.at[src_v.at[g + 3]], rows_b, gsem_b)
            return carry

        lax.fori_loop(0, _NCHH // 2, body, 0)
        pltpu.make_async_copy(xs_hbm.at[src_v.at[0]], rows_a, gsem_a).wait()
        pltpu.make_async_copy(xs_hbm.at[src_v.at[1]], rows_b, gsem_b).wait()

    plsc.subcore_barrier()
    pltpu.sync_copy(acc.at[pl.ds(r0, _RPT)], out_hbm.at[c, pl.ds(r0, _RPT)])


def _pre_body(deg_ref, x_ref, dinv_ref, xs_ref):
    d = deg_ref[:, 0] + deg_ref[:, 1]
    dinv = jnp.where(d > 0, lax.rsqrt(jnp.maximum(d, 1e-12)), 0.0)
    dinv_ref[:, 0] = dinv
    xs_ref[...] = x_ref[...] * dinv[:, None]


def _layer_body(wb_ref, s_ref, dinv_ref, x_ref, xs_ref):
    w = wb_ref[0]
    b = wb_ref[1]
    dinv = dinv_ref[:, 0]
    x = jnp.maximum((s_ref[0] + s_ref[1]) * dinv[:, None] * w + b, 0.0)
    x_ref[...] = x
    xs_ref[...] = x * dinv[:, None]


def _final_body(wb_ref, s_ref, dinv_ref, ro_ref, x_ref, out3_ref):
    w = wb_ref[0]
    b = wb_ref[1]
    dinv = dinv_ref[:, 0]
    x = jnp.maximum((s_ref[0] + s_ref[1]) * dinv[:, None] * w + b, 0.0)
    x_ref[...] = x
    out3_ref[...] = (x[None, :, :] * ro_ref[0][:, None, None]
                     + ro_ref[1][:, None, None])


def kernel(x, edge_index, conv_W, conv_b, ro_W, ro_b):
    f32 = jnp.float32
    pad = (jnp.arange(_TOTAL - _E, dtype=jnp.int32) % _NPAD_ROWS) + _N
    src_flat = jnp.concatenate([edge_index[0], pad])
    dst_flat = jnp.concatenate([edge_index[1], pad])
    src_p = src_flat.reshape(_NT, _NPH, _NCHH, _K)
    dst_p = dst_flat.reshape(_NT, _NPH, _NCHH, _K)
    ring_pad = ((jnp.arange(_NT * _NPH * 2 * _K, dtype=jnp.int32)
                 % _NPAD_ROWS) + _N).reshape(_NT, _NPH, 2, _K)
    src_p = jnp.concatenate([src_p, ring_pad], axis=2)
    dst_deg = dst_flat.reshape(_NT, _NCHD, _KD)
    x_pad = jnp.zeros((_NP, _C), f32).at[:_N].set(x)
    zeros_big = jnp.zeros((_NP, _C), f32)
    ones_np = jnp.ones((_NP,), f32)
    deg_init = jnp.stack(
        [jnp.concatenate([jnp.ones((_N,), f32), jnp.zeros((_NPAD_ROWS,), f32)]),
         jnp.zeros((_NP,), f32)])
    wb = jnp.stack([conv_W[0, 0], conv_b[0]])
    ro = jnp.concatenate([ro_W, ro_b[None, :]], axis=0)

    deg2 = _deg_kernel(dst_deg, deg_init, ones_np)
    deg_t = deg2.T

    grid = (_NP // _BN,)
    dinv, xs = pl.pallas_call(
        _pre_body,
        grid=grid,
        in_specs=[
            pl.BlockSpec((_BN, 2), lambda i: (i, 0)),
            pl.BlockSpec((_BN, _C), lambda i: (i, 0)),
        ],
        out_specs=[
            pl.BlockSpec((_BN, 1), lambda i: (i, 0)),
            pl.BlockSpec((_BN, _C), lambda i: (i, 0)),
        ],
        out_shape=[
            jax.ShapeDtypeStruct((_NP, 1), f32),
            jax.ShapeDtypeStruct((_NP, _C), f32),
        ],
    )(deg_t, x_pad)

    layer_call = pl.pallas_call(
        _layer_body,
        grid=grid,
        in_specs=[
            pl.BlockSpec(memory_space=pltpu.SMEM),
            pl.BlockSpec((2, _BN, _C), lambda i: (0, i, 0)),
            pl.BlockSpec((_BN, 1), lambda i: (i, 0)),
        ],
        out_specs=[
            pl.BlockSpec((_BN, _C), lambda i: (i, 0)),
            pl.BlockSpec((_BN, _C), lambda i: (i, 0)),
        ],
        out_shape=[
            jax.ShapeDtypeStruct((_NP, _C), f32),
            jax.ShapeDtypeStruct((_NP, _C), f32),
        ],
    )

    xs_list = []
    for _ in range(_NLAYERS - 1):
        s_part = _spmm_kernel(xs, zeros_big, src_p, dst_p)
        xlayer, xs = layer_call(wb, s_part, dinv)
        xs_list.append(xlayer)

    s_part = _spmm_kernel(xs, zeros_big, src_p, dst_p)
    x_last, out3 = pl.pallas_call(
        _final_body,
        grid=grid,
        in_specs=[
            pl.BlockSpec(memory_space=pltpu.SMEM),
            pl.BlockSpec((2, _BN, _C), lambda i: (0, i, 0)),
            pl.BlockSpec((_BN, 1), lambda i: (i, 0)),
            pl.BlockSpec((2, _NCLASS), lambda i: (0, 0)),
        ],
        out_specs=[
            pl.BlockSpec((_BN, _C), lambda i: (i, 0)),
            pl.BlockSpec((_NCLASS, _BN, _C), lambda i: (0, i, 0)),
        ],
        out_shape=[
            jax.ShapeDtypeStruct((_NP, _C), f32),
            jax.ShapeDtypeStruct((_NCLASS, _NP, _C), f32),
        ],
    )(wb, s_part, dinv, ro)
    xs_list.append(x_last)

    out = jnp.transpose(out3, (1, 2, 0))[:_N]
    x_all = jnp.stack([x] + [xl[:_N] for xl in xs_list], axis=1)
    return (out, x_all)
